# Initial kernel scaffold; baseline (speedup 1.0000x reference)
import jax
import jax.numpy as jnp
from jax.experimental import pallas as pl


def kernel(x, edge_index, BU_edge_index, root, rootindex, batch, W_td1, b_td1, W_td2, b_td2, W_bu1, b_bu1, W_bu2, b_bu2, W_fc, b_fc):
    raise NotImplementedError("write your pallas kernel here")


# speedup vs baseline: 14.2768x; 14.2768x over previous
import functools
import jax
import jax.numpy as jnp
from jax import lax
from jax.experimental import pallas as pl
from jax.experimental.pallas import tpu as pltpu
from jax.experimental.pallas import tpu_sc as plsc

N = 10000
NPAD = 10240
E = 320000
B = 128
F = 128
FH = F // 2
NCORE = 2
NSUB = 16
CH = 128
NCHUNK = 157
EPT = NCHUNK * CH
ROWS_PT = NPAD // NSUB
DUMMY = NPAD - 1
RB = 1280
GRID = NPAD // RB

_f32 = jnp.float32


@functools.cache
def _get_mesh():
    return plsc.VectorSubcoreMesh(
        core_axis_name="c", subcore_axis_name="s", num_cores=NCORE,
        num_subcores=NSUB)


def _deg_body(dst_td, dst_bu, deg_td, deg_bu, idx_v, ones_v, zb_v, hist):
    c = lax.axis_index("c")
    s = lax.axis_index("s")
    for k in range(CH // 16):
        ones_v[pl.ds(k * 16, 16)] = jnp.ones((16,), _f32)
    for k in range(ROWS_PT // 16):
        zb_v[pl.ds(k * 16, 16)] = jnp.zeros((16,), _f32)
    sl = pl.ds(s * ROWS_PT, ROWS_PT)
    pltpu.sync_copy(zb_v, hist.at[sl])
    plsc.subcore_barrier()

    def run(dst_hbm, out_hbm):
        pltpu.sync_copy(dst_hbm.at[s], idx_v)

        def body(j, carry):
            pltpu.sync_copy(ones_v, hist.at[idx_v.at[j]], add=True)
            return carry

        lax.fori_loop(0, NCHUNK, body, 0)
        plsc.subcore_barrier()
        pltpu.sync_copy(hist.at[sl], out_hbm.at[sl])

    @pl.when(c == 0)
    def _():
        run(dst_td, deg_td)

    @pl.when(c == 1)
    def _():
        run(dst_bu, deg_bu)


@functools.cache
def _get_deg_kernel():
    return pl.kernel(
        _deg_body,
        out_type=[jax.ShapeDtypeStruct((NPAD,), _f32),
                  jax.ShapeDtypeStruct((NPAD,), _f32)],
        mesh=_get_mesh(),
        scratch_types=[
            pltpu.VMEM((NCHUNK, CH), jnp.int32),
            pltpu.VMEM((CH,), _f32),
            pltpu.VMEM((ROWS_PT,), _f32),
            pltpu.VMEM_SHARED((NPAD,), _f32),
        ],
    )


def _mp_body(h_td0, h_td1, h_bu0, h_bu1, src_td, dst_td, src_bu, dst_bu,
             zeros_hbm, a_td0, a_td1, a_bu0, a_bu1,
             isrc_v, idst_v, rows_v, sem, acc):
    c = lax.axis_index("c")
    s = lax.axis_index("s")
    sl = pl.ds(s * ROWS_PT, ROWS_PT)

    def run(h_hbm, src_hbm, dst_hbm, out_hbm):
        pltpu.sync_copy(zeros_hbm.at[sl], acc.at[sl])
        pltpu.sync_copy(src_hbm.at[s], isrc_v)
        pltpu.sync_copy(dst_hbm.at[s], idst_v)
        plsc.subcore_barrier()

        def body(j, carry):
            pltpu.async_copy(h_hbm.at[isrc_v.at[j]], rows_v, sem).wait()
            pltpu.sync_copy(rows_v, acc.at[idst_v.at[j]], add=True)
            return carry

        lax.fori_loop(0, NCHUNK, body, 0)
        plsc.subcore_barrier()
        pltpu.sync_copy(acc.at[sl], out_hbm.at[sl])
        plsc.subcore_barrier()

    @pl.when(c == 0)
    def _():
        run(h_td0, src_td, dst_td, a_td0)
        run(h_bu0, src_bu, dst_bu, a_bu0)

    @pl.when(c == 1)
    def _():
        run(h_td1, src_td, dst_td, a_td1)
        run(h_bu1, src_bu, dst_bu, a_bu1)


@functools.cache
def _get_mp_kernel():
    half = jax.ShapeDtypeStruct((NPAD, FH), _f32)
    return pl.kernel(
        _mp_body,
        out_type=[half, half, half, half],
        mesh=_get_mesh(),
        scratch_types=[
            pltpu.VMEM((NCHUNK, CH), jnp.int32),
            pltpu.VMEM((NCHUNK, CH), jnp.int32),
            pltpu.VMEM((CH, FH), _f32),
            pltpu.SemaphoreType.DMA,
            pltpu.VMEM_SHARED((NPAD, FH), _f32),
        ],
        compiler_params=pltpu.CompilerParams(use_tc_tiling_on_sc=False),
    )


def _tc1_body(x_ref, degt_ref, degb_ref, wt_ref, wb_ref,
              hpt0_ref, hpt1_ref, hpb0_ref, hpb1_ref, dit_ref, dib_ref):
    x = x_ref[...]
    dit = lax.rsqrt(degt_ref[...] + 1.0)
    dib = lax.rsqrt(degb_ref[...] + 1.0)
    dit_ref[...] = dit
    dib_ref[...] = dib
    hpt = dit * jnp.dot(x, wt_ref[...], preferred_element_type=_f32)
    hpb = dib * jnp.dot(x, wb_ref[...], preferred_element_type=_f32)
    hpt0_ref[...] = hpt[:, :FH]
    hpt1_ref[...] = hpt[:, FH:]
    hpb0_ref[...] = hpb[:, :FH]
    hpb1_ref[...] = hpb[:, FH:]


def _tc2_body(at0_ref, at1_ref, hpt0_ref, hpt1_ref, dit_ref, b1t_ref,
              w2ta_ref, w2tb_ref,
              ab0_ref, ab1_ref, hpb0_ref, hpb1_ref, dib_ref, b1b_ref,
              w2ba_ref, w2bb_ref,
              root_ref, batch_ref,
              x2t_ref, hp2t0_ref, hp2t1_ref, x2b_ref, hp2b0_ref, hp2b1_ref):
    oh = (batch_ref[...] == lax.broadcasted_iota(jnp.int32, (RB, B), 1)
          ).astype(_f32)
    rootext = jnp.dot(oh, root_ref[...], preferred_element_type=_f32)
    v = jnp.maximum(rootext, 0.0)

    def branch(a0_ref, a1_ref, hp0_ref, hp1_ref, di_ref, b1_ref,
               w2a_ref, w2b_ref, x2_ref, hp20_ref, hp21_ref):
        di = di_ref[...]
        ah = jnp.concatenate(
            [a0_ref[...] + hp0_ref[...], a1_ref[...] + hp1_ref[...]], axis=1)
        x2 = di * ah + b1_ref[...]
        x2_ref[...] = x2
        u = jnp.maximum(x2, 0.0)
        h2 = (jnp.dot(u, w2a_ref[...], preferred_element_type=_f32)
              + jnp.dot(v, w2b_ref[...], preferred_element_type=_f32))
        hp2 = di * h2
        hp20_ref[...] = hp2[:, :FH]
        hp21_ref[...] = hp2[:, FH:]

    branch(at0_ref, at1_ref, hpt0_ref, hpt1_ref, dit_ref, b1t_ref,
           w2ta_ref, w2tb_ref, x2t_ref, hp2t0_ref, hp2t1_ref)
    branch(ab0_ref, ab1_ref, hpb0_ref, hpb1_ref, dib_ref, b1b_ref,
           w2ba_ref, w2bb_ref, x2b_ref, hp2b0_ref, hp2b1_ref)


def _tc3_body(a2t0_ref, a2t1_ref, hp2t0_ref, hp2t1_ref, dit_ref, b2t_ref,
              x2t_ref,
              a2b0_ref, a2b1_ref, hp2b0_ref, hp2b1_ref, dib_ref, b2b_ref,
              x2b_ref,
              batch_ref, ri_ref, wfc_ref, bfc_ref, out_ref,
              sum_t, sum_b, cnt_s, xr_t, xr_b):
    pid = pl.program_id(0)
    oh = (batch_ref[...] == lax.broadcasted_iota(jnp.int32, (RB, B), 1)
          ).astype(_f32)
    row_ids = pid * RB + lax.broadcasted_iota(jnp.int32, (B, RB), 1)
    ohr = (ri_ref[...] == row_ids).astype(_f32)

    def seg(a20_ref, a21_ref, hp20_ref, hp21_ref, di_ref, b2_ref, x2_ref,
            sum_s, xr_s):
        ah = jnp.concatenate(
            [a20_ref[...] + hp20_ref[...], a21_ref[...] + hp21_ref[...]],
            axis=1)
        out2 = di_ref[...] * ah + b2_ref[...]
        r = jnp.maximum(out2, 0.0)
        psum = lax.dot_general(oh, r, (((0,), (0,)), ((), ())),
                               preferred_element_type=_f32)
        pxr = jnp.dot(ohr, x2_ref[...], preferred_element_type=_f32)

        @pl.when(pid == 0)
        def _():
            sum_s[...] = psum
            xr_s[...] = pxr

        @pl.when(pid != 0)
        def _():
            sum_s[...] = sum_s[...] + psum
            xr_s[...] = xr_s[...] + pxr

    seg(a2t0_ref, a2t1_ref, hp2t0_ref, hp2t1_ref, dit_ref, b2t_ref, x2t_ref,
        sum_t, xr_t)
    seg(a2b0_ref, a2b1_ref, hp2b0_ref, hp2b1_ref, dib_ref, b2b_ref, x2b_ref,
        sum_b, xr_b)
    pcnt = lax.dot_general(oh, jnp.ones((RB, F), _f32),
                           (((0,), (0,)), ((), ())),
                           preferred_element_type=_f32)

    @pl.when(pid == 0)
    def _():
        cnt_s[...] = pcnt

    @pl.when(pid != 0)
    def _():
        cnt_s[...] = cnt_s[...] + pcnt

    @pl.when(pid == GRID - 1)
    def _():
        cnt = cnt_s[...]
        denom = jnp.maximum(cnt, 1.0)
        mean_t = sum_t[...] / denom
        mean_b = sum_b[...] / denom
        xrt = jnp.where(cnt > 0.0, xr_t[...], 0.0)
        xrb = jnp.where(cnt > 0.0, xr_b[...], 0.0)
        w = wfc_ref[...]
        logits = (jnp.dot(mean_b, w[0], preferred_element_type=_f32)
                  + jnp.dot(xrb, w[1], preferred_element_type=_f32)
                  + jnp.dot(mean_t, w[2], preferred_element_type=_f32)
                  + jnp.dot(xrt, w[3], preferred_element_type=_f32)
                  + bfc_ref[...])
        col = lax.broadcasted_iota(jnp.int32, (B, F), 1)
        valid = col < 4
        lm = jnp.where(valid, logits, -jnp.inf)
        m = jnp.max(lm, axis=1, keepdims=True)
        e = jnp.where(valid, jnp.exp(lm - m), 0.0)
        lse = jnp.log(jnp.sum(e, axis=1, keepdims=True))
        out_ref[...] = logits - m - lse


def _pad_edges(ei):
    src = jnp.concatenate(
        [ei[0], jnp.zeros((NSUB * EPT - E,), jnp.int32)])
    dst = jnp.concatenate(
        [ei[1], jnp.full((NSUB * EPT - E,), DUMMY, jnp.int32)])
    return (src.reshape(NSUB, NCHUNK, CH), dst.reshape(NSUB, NCHUNK, CH))


def kernel(x, edge_index, BU_edge_index, root, rootindex, batch,
           W_td1, b_td1, W_td2, b_td2, W_bu1, b_bu1, W_bu2, b_bu2,
           W_fc, b_fc):
    x_p = jnp.pad(x, ((0, NPAD - N), (0, 0)))
    batch_p = jnp.pad(batch, (0, NPAD - N), constant_values=B).reshape(
        NPAD, 1)
    ri_col = rootindex.reshape(B, 1)
    src_td, dst_td = _pad_edges(edge_index)
    src_bu, dst_bu = _pad_edges(BU_edge_index)
    zeros_half = jnp.zeros((NPAD, FH), _f32)
    wfc = W_fc.reshape(4, F, 4)
    wfc = jnp.pad(wfc, ((0, 0), (0, 0), (0, F - 4)))
    bfc = jnp.pad(b_fc, (0, F - 4)).reshape(1, F)

    deg_td, deg_bu = _get_deg_kernel()(dst_td, dst_bu)
    deg_td = deg_td.reshape(NPAD, 1)
    deg_bu = deg_bu.reshape(NPAD, 1)

    rowspec = pl.BlockSpec((RB, F), lambda i: (i, 0))
    halfspec = pl.BlockSpec((RB, FH), lambda i: (i, 0))
    colspec = pl.BlockSpec((RB, 1), lambda i: (i, 0))
    wspec = pl.BlockSpec((F, F), lambda i: (0, 0))
    bspec = pl.BlockSpec((1, F), lambda i: (0, 0))
    rowout = jax.ShapeDtypeStruct((NPAD, F), _f32)
    halfout = jax.ShapeDtypeStruct((NPAD, FH), _f32)
    colout = jax.ShapeDtypeStruct((NPAD, 1), _f32)

    hpt0, hpt1, hpb0, hpb1, dinv_td, dinv_bu = pl.pallas_call(
        _tc1_body,
        grid=(GRID,),
        in_specs=[rowspec, colspec, colspec, wspec, wspec],
        out_specs=[halfspec, halfspec, halfspec, halfspec, colspec, colspec],
        out_shape=[halfout, halfout, halfout, halfout, colout, colout],
    )(x_p, deg_td, deg_bu, W_td1, W_bu1)

    at0, at1, ab0, ab1 = _get_mp_kernel()(
        hpt0, hpt1, hpb0, hpb1, src_td, dst_td, src_bu, dst_bu, zeros_half)

    b1t = b_td1.reshape(1, F)
    b1b = b_bu1.reshape(1, F)
    x2_td, hp2t0, hp2t1, x2_bu, hp2b0, hp2b1 = pl.pallas_call(
        _tc2_body,
        grid=(GRID,),
        in_specs=[halfspec, halfspec, halfspec, halfspec, colspec, bspec,
                  wspec, wspec,
                  halfspec, halfspec, halfspec, halfspec, colspec, bspec,
                  wspec, wspec,
                  wspec, colspec],
        out_specs=[rowspec, halfspec, halfspec, rowspec, halfspec, halfspec],
        out_shape=[rowout, halfout, halfout, rowout, halfout, halfout],
    )(at0, at1, hpt0, hpt1, dinv_td, b1t, W_td2[:F], W_td2[F:],
      ab0, ab1, hpb0, hpb1, dinv_bu, b1b, W_bu2[:F], W_bu2[F:],
      root, batch_p)

    a2t0, a2t1, a2b0, a2b1 = _get_mp_kernel()(
        hp2t0, hp2t1, hp2b0, hp2b1, src_td, dst_td, src_bu, dst_bu,
        zeros_half)

    b2t = b_td2.reshape(1, F)
    b2b = b_bu2.reshape(1, F)
    out_pad = pl.pallas_call(
        _tc3_body,
        grid=(GRID,),
        in_specs=[halfspec, halfspec, halfspec, halfspec, colspec, bspec,
                  rowspec,
                  halfspec, halfspec, halfspec, halfspec, colspec, bspec,
                  rowspec,
                  colspec,
                  pl.BlockSpec((B, 1), lambda i: (0, 0)),
                  pl.BlockSpec((4, F, F), lambda i: (0, 0, 0)),
                  bspec],
        out_specs=pl.BlockSpec((B, F), lambda i: (0, 0)),
        out_shape=jax.ShapeDtypeStruct((B, F), _f32),
        scratch_shapes=[pltpu.VMEM((B, F), _f32)] * 5,
    )(a2t0, a2t1, hp2t0, hp2t1, dinv_td, b2t, x2_td,
      a2b0, a2b1, hp2b0, hp2b1, dinv_bu, b2b, x2_bu,
      batch_p, ri_col, wfc, bfc)

    return out_pad[:, :4]
